# Initial kernel scaffold; baseline (speedup 1.0000x reference)
import jax
import jax.numpy as jnp
from jax.experimental import pallas as pl


def kernel(protein_pos, protein_atom_feature, ligand_pos, ligand_atom_feature, batch_protein, batch_ligand, output_kind, enc_ligand_feature, enc_node_feature, enc_graph_feature, params):
    raise NotImplementedError("write your pallas kernel here")


# speedup vs baseline: 5.9802x; 5.9802x over previous
import functools

import jax
import jax.numpy as jnp
from jax import lax
from jax.experimental import pallas as pl
from jax.experimental.pallas import tpu as pltpu
from jax.experimental.pallas import tpu_sc as plsc

H = 128
K = 16
NUM_RBF = 20
CUTOFF = 10.0
NGRAPH = 32
N_PROT = 6000
N_LIG = 600
N_NODES = N_PROT + N_LIG
BN = 128
NPAD = 6656
NBLK = NPAD // BN
EPAD = NPAD * K
OUT_DIM = 2

_RBF_STEP = CUTOFF / (NUM_RBF - 1)
_RBF_COEFF = -0.5 / _RBF_STEP ** 2

_pcall = pl.pallas_call


def _mm_kernel(x_ref, w_ref, o_ref):
    o_ref[...] = jnp.dot(x_ref[...], w_ref[...],
                         preferred_element_type=jnp.float32)


def _proj(x, w):
    return _pcall(
        _mm_kernel,
        grid=(NBLK,),
        in_specs=[pl.BlockSpec((BN, x.shape[1]), lambda i: (i, 0)),
                  pl.BlockSpec(w.shape, lambda i: (0, 0))],
        out_specs=pl.BlockSpec((BN, H), lambda i: (i, 0)),
        out_shape=jax.ShapeDtypeStruct((NPAD, H), jnp.float32),
    )(x, w)


def _two_mm_kernel(x_ref, wa_ref, wb_ref, oa_ref, ob_ref):
    x = x_ref[...]
    oa_ref[...] = jnp.dot(x, wa_ref[...], preferred_element_type=jnp.float32)
    ob_ref[...] = jnp.dot(x, wb_ref[...], preferred_element_type=jnp.float32)


def _two_mm(h, wa, wb):
    return _pcall(
        _two_mm_kernel,
        grid=(NBLK,),
        in_specs=[pl.BlockSpec((BN, H), lambda i: (i, 0)),
                  pl.BlockSpec((H, H), lambda i: (0, 0)),
                  pl.BlockSpec((H, H), lambda i: (0, 0))],
        out_specs=[pl.BlockSpec((BN, H), lambda i: (i, 0)),
                   pl.BlockSpec((BN, H), lambda i: (i, 0))],
        out_shape=[jax.ShapeDtypeStruct((NPAD, H), jnp.float32),
                   jax.ShapeDtypeStruct((NPAD, H), jnp.float32)],
    )(h, wa, wb)


def _knn_kernel(pd_ref, psT_ref, bd_ref, bs_ref, idx_ref, d_ref):
    i = pl.program_id(0)
    pd = pd_ref[...]
    psT = psT_ref[...]
    d2 = (jnp.sum(pd * pd, axis=1, keepdims=True)
          + jnp.sum(psT * psT, axis=0, keepdims=True)
          - 2.0 * jnp.dot(pd, psT, preferred_element_type=jnp.float32))
    src_iota = lax.broadcasted_iota(jnp.int32, (BN, NPAD), 1)
    dst_ids = i * BN + lax.broadcasted_iota(jnp.int32, (BN, NPAD), 0)
    bad = (bd_ref[...] != bs_ref[...]) | (dst_ids == src_iota)
    d2 = jnp.where(bad, jnp.inf, d2)
    for t in range(K):
        mn = jnp.min(d2, axis=1, keepdims=True)
        cand = jnp.where(d2 == mn, src_iota, NPAD + 1)
        sel = jnp.min(cand, axis=1, keepdims=True)
        idx_ref[:, t:t + 1] = sel
        d_ref[:, t:t + 1] = jnp.sqrt(jnp.maximum(mn, 0.0) + 1e-12)
        d2 = jnp.where(src_iota == sel, jnp.inf, d2)


def _knn(pos, psT, bcol, brow):
    return _pcall(
        _knn_kernel,
        grid=(NBLK,),
        in_specs=[pl.BlockSpec((BN, 3), lambda i: (i, 0)),
                  pl.BlockSpec((3, NPAD), lambda i: (0, 0)),
                  pl.BlockSpec((BN, 1), lambda i: (i, 0)),
                  pl.BlockSpec((1, NPAD), lambda i: (0, 0))],
        out_specs=[pl.BlockSpec((BN, K), lambda i: (i, 0)),
                   pl.BlockSpec((BN, K), lambda i: (i, 0))],
        out_shape=[jax.ShapeDtypeStruct((NPAD, K), jnp.int32),
                   jax.ShapeDtypeStruct((NPAD, K), jnp.float32)],
    )(pos, psT, bcol, brow)


_SC_NC = 2
_SC_NW = 32
_SC_CH = 128
_PER_W = EPAD // _SC_NW
_SC_NCH = _PER_W // _SC_CH


@functools.cache
def _build_sc_gather():
    @functools.partial(
        pl.kernel,
        mesh=plsc.VectorSubcoreMesh(core_axis_name="c", subcore_axis_name="s"),
        out_type=jax.ShapeDtypeStruct((EPAD, H), jnp.float32),
        scratch_types=[
            pltpu.VMEM((_SC_CH,), jnp.int32),
            pltpu.VMEM((_SC_CH, H), jnp.float32),
            pltpu.SemaphoreType.DMA,
        ],
    )
    def _sc_gather(table_hbm, idx_hbm, out_hbm, idx_v, rows_v, sem):
        wid = lax.axis_index("s") * _SC_NC + lax.axis_index("c")
        base = wid * _PER_W

        def body(i, carry):
            off = base + i * _SC_CH
            pltpu.sync_copy(idx_hbm.at[pl.ds(off, _SC_CH)], idx_v)
            pltpu.async_copy(table_hbm.at[idx_v], rows_v, sem).wait()
            pltpu.sync_copy(rows_v, out_hbm.at[pl.ds(off, _SC_CH)])
            return carry

        lax.fori_loop(0, _SC_NCH, body, 0)

    return _sc_gather


def _GATHER(table, idx):
    return _build_sc_gather()(table, idx)


def _edge_kernel(h_ref, hA_ref, g_ref, d_ref,
                 c_ref, be1_ref, w2_ref, be2_ref,
                 wn1h_ref, wn1a_ref, bn1_ref, wn2_ref, bn2_ref,
                 lng_ref, lnb_ref, o_ref):
    d = d_ref[...]
    off = lax.broadcasted_iota(jnp.int32, (1, NUM_RBF), 1).astype(
        jnp.float32) * _RBF_STEP
    rbf = jnp.exp(_RBF_COEFF * (d - off) ** 2)
    cut = (d <= CUTOFF).astype(jnp.float32)
    hA = hA_ref[...]
    hA_e = jnp.reshape(jnp.broadcast_to(hA[:, None, :], (BN, K, H)),
                       (BN * K, H))
    m1 = jnp.maximum(
        hA_e + g_ref[...]
        + jnp.dot(rbf, c_ref[...], preferred_element_type=jnp.float32)
        + be1_ref[...], 0.0)
    m = jnp.maximum(
        jnp.dot(m1, w2_ref[...], preferred_element_type=jnp.float32)
        + be2_ref[...], 0.0) * cut
    agg = jnp.sum(jnp.reshape(m, (BN, K, H)), axis=1)
    h = h_ref[...]
    u1 = jnp.maximum(
        jnp.dot(h, wn1h_ref[...], preferred_element_type=jnp.float32)
        + jnp.dot(agg, wn1a_ref[...], preferred_element_type=jnp.float32)
        + bn1_ref[...], 0.0)
    u = jnp.dot(u1, wn2_ref[...], preferred_element_type=jnp.float32) \
        + bn2_ref[...]
    r = h + u
    mu = jnp.mean(r, axis=1, keepdims=True)
    var = jnp.mean((r - mu) ** 2, axis=1, keepdims=True)
    o_ref[...] = (r - mu) * lax.rsqrt(var + 1e-5) * lng_ref[...] \
        + lnb_ref[...]


def _edge_node(h, hA, g, dcol, c, be1, w2, be2,
               wn1h, wn1a, bn1, wn2, bn2, lng, lnb):
    full = lambda a: pl.BlockSpec(a.shape, lambda i: (0, 0))
    return _pcall(
        _edge_kernel,
        grid=(NBLK,),
        in_specs=[pl.BlockSpec((BN, H), lambda i: (i, 0)),
                  pl.BlockSpec((BN, H), lambda i: (i, 0)),
                  pl.BlockSpec((BN * K, H), lambda i: (i, 0)),
                  pl.BlockSpec((BN * K, 1), lambda i: (i, 0)),
                  full(c), full(be1), full(w2), full(be2),
                  full(wn1h), full(wn1a), full(bn1), full(wn2), full(bn2),
                  full(lng), full(lnb)],
        out_specs=pl.BlockSpec((BN, H), lambda i: (i, 0)),
        out_shape=jax.ShapeDtypeStruct((NPAD, H), jnp.float32),
    )(h, hA, g, dcol, c, be1, w2, be2, wn1h, wn1a, bn1, wn2, bn2, lng, lnb)


def _final_kernel(h_ref, enc_ref, brow_ref, w1h_ref, w1e_ref, b1_ref,
                  w2_ref, b2_ref, wo1h_ref, wo1g_ref, bo1_ref,
                  wo2_ref, bo2_ref, encg_ref, kind_ref, o_ref, acc_ref):
    i = pl.program_id(0)
    x1 = jnp.maximum(
        jnp.dot(h_ref[...], w1h_ref[...], preferred_element_type=jnp.float32)
        + jnp.dot(enc_ref[...], w1e_ref[...],
                  preferred_element_type=jnp.float32)
        + b1_ref[...], 0.0)
    x2 = jnp.dot(x1, w2_ref[...], preferred_element_type=jnp.float32) \
        + b2_ref[...]
    s_iota = lax.broadcasted_iota(jnp.int32, (NGRAPH, BN), 0)
    oh = (brow_ref[...] == s_iota).astype(jnp.float32)
    part = jnp.dot(oh, x2, preferred_element_type=jnp.float32)

    @pl.when(i == 0)
    def _():
        acc_ref[...] = jnp.zeros_like(acc_ref)

    acc_ref[...] += part

    @pl.when(i == pl.num_programs(0) - 1)
    def _():
        pre = acc_ref[...]
        z = (jnp.dot(pre, wo1h_ref[...], preferred_element_type=jnp.float32)
             + jnp.dot(encg_ref[...], wo1g_ref[...],
                       preferred_element_type=jnp.float32)
             + bo1_ref[...])
        sp = jnp.maximum(z, 0.0) + jnp.log1p(jnp.exp(-jnp.abs(z))) \
            - jnp.log(2.0).astype(jnp.float32)
        o2 = jnp.dot(sp, wo2_ref[...], preferred_element_type=jnp.float32) \
            + bo2_ref[...]
        k_oh = (kind_ref[...] - 1
                == lax.broadcasted_iota(jnp.int32, (NGRAPH, OUT_DIM), 1)
                ).astype(jnp.float32)
        o_ref[...] = jnp.sum(o2 * k_oh, axis=1, keepdims=True)


def _final(h, enc_pad, brow, w1h, w1e, b1, w2, b2,
           wo1h, wo1g, bo1, wo2, bo2, encg, kind):
    full = lambda a: pl.BlockSpec(a.shape, lambda i: (0, 0))
    return _pcall(
        _final_kernel,
        grid=(NBLK,),
        in_specs=[pl.BlockSpec((BN, H), lambda i: (i, 0)),
                  pl.BlockSpec((BN, enc_pad.shape[1]), lambda i: (i, 0)),
                  pl.BlockSpec((1, BN), lambda i: (0, i)),
                  full(w1h), full(w1e), full(b1), full(w2), full(b2),
                  full(wo1h), full(wo1g), full(bo1), full(wo2), full(bo2),
                  full(encg), full(kind)],
        out_specs=pl.BlockSpec((NGRAPH, 1), lambda i: (0, 0)),
        out_shape=jax.ShapeDtypeStruct((NGRAPH, 1), jnp.float32),
        scratch_shapes=[pltpu.VMEM((NGRAPH, H), jnp.float32)],
    )(h, enc_pad, brow, w1h, w1e, b1, w2, b2,
      wo1h, wo1g, bo1, wo2, bo2, encg, kind)


def kernel(protein_pos, protein_atom_feature, ligand_pos, ligand_atom_feature,
           batch_protein, batch_ligand, output_kind,
           enc_ligand_feature, enc_node_feature, enc_graph_feature, params):
    f32 = jnp.float32
    lig_feat = jnp.concatenate(
        [ligand_atom_feature, enc_ligand_feature], axis=1)
    pfd = protein_atom_feature.shape[1]
    lfd = lig_feat.shape[1]
    xa_d = 64
    xa = jnp.zeros((NPAD, xa_d), f32)
    xa = xa.at[:N_PROT, :pfd].set(protein_atom_feature)
    xa = xa.at[N_PROT:N_NODES, pfd:pfd + lfd].set(lig_feat)
    xa = xa.at[:N_PROT, pfd + lfd].set(1.0)
    xa = xa.at[N_PROT:N_NODES, pfd + lfd + 1].set(1.0)
    wa = jnp.zeros((xa_d, H), f32)
    wa = wa.at[:pfd].set(params['Wp'])
    wa = wa.at[pfd:pfd + lfd].set(params['Wl'])
    wa = wa.at[pfd + lfd].set(params['bp'])
    wa = wa.at[pfd + lfd + 1].set(params['bl'])

    pos = jnp.concatenate(
        [protein_pos, ligand_pos,
         jnp.zeros((NPAD - N_NODES, 3), f32)], axis=0)
    batch_all = jnp.concatenate(
        [batch_protein, batch_ligand]).astype(jnp.int32)
    batch_pad = jnp.concatenate(
        [batch_all, jnp.full((NPAD - N_NODES,), -1, jnp.int32)])
    brow = batch_pad.reshape(1, NPAD)
    bcol = batch_pad.reshape(NPAD, 1)
    psT = pos.T

    order = jnp.argsort(batch_all)
    inv = jnp.argsort(order)
    enc_aligned = enc_node_feature[inv]
    enc_pad = jnp.concatenate(
        [enc_aligned,
         jnp.zeros((NPAD - N_NODES, enc_aligned.shape[1]), f32)], axis=0)

    h = _proj(xa, wa)
    idx, dsel = _knn(pos, psT, bcol, brow)
    idx_flat = idx.reshape(EPAD)
    dcol = dsel.reshape(EPAD, 1)

    for lp in params['enc_layers']:
        a_w = lp['We1'][:H]
        b_w = lp['We1'][H:2 * H]
        c_w = lp['We1'][2 * H:]
        hA, hB = _two_mm(h, a_w, b_w)
        g = _GATHER(hB, idx_flat)
        h = _edge_node(h, hA, g, dcol, c_w,
                       lp['be1'].reshape(1, H), lp['We2'],
                       lp['be2'].reshape(1, H),
                       lp['Wn1'][:H], lp['Wn1'][H:],
                       lp['bn1'].reshape(1, H), lp['Wn2'],
                       lp['bn2'].reshape(1, H),
                       lp['ln_g'].reshape(1, H), lp['ln_b'].reshape(1, H))

    out = _final(
        h, enc_pad, brow,
        params['Wenc1'][:H], params['Wenc1'][H:],
        params['benc1'].reshape(1, H),
        params['Wenc2'], params['benc2'].reshape(1, H),
        params['Wo1'][:H], params['Wo1'][H:],
        params['bo1'].reshape(1, H),
        params['Wo2'], params['bo2'].reshape(1, OUT_DIM),
        enc_graph_feature.astype(f32),
        output_kind.reshape(NGRAPH, 1).astype(jnp.int32))
    return out


# speedup vs baseline: 6.2726x; 1.0489x over previous
import functools

import jax
import jax.numpy as jnp
from jax import lax
from jax.experimental import pallas as pl
from jax.experimental.pallas import tpu as pltpu
from jax.experimental.pallas import tpu_sc as plsc

H = 128
K = 16
NUM_RBF = 20
CUTOFF = 10.0
NGRAPH = 32
N_PROT = 6000
N_LIG = 600
N_NODES = N_PROT + N_LIG
BN = 128
NPAD = 6656
NBLK = NPAD // BN
EPAD = NPAD * K
OUT_DIM = 2

_RBF_STEP = CUTOFF / (NUM_RBF - 1)
_RBF_COEFF = -0.5 / _RBF_STEP ** 2

_pcall = pl.pallas_call


def _mm_kernel(x_ref, w_ref, o_ref):
    o_ref[...] = jnp.dot(x_ref[...], w_ref[...],
                         preferred_element_type=jnp.float32)


def _proj(x, w):
    return _pcall(
        _mm_kernel,
        grid=(NBLK,),
        in_specs=[pl.BlockSpec((BN, x.shape[1]), lambda i: (i, 0)),
                  pl.BlockSpec(w.shape, lambda i: (0, 0))],
        out_specs=pl.BlockSpec((BN, H), lambda i: (i, 0)),
        out_shape=jax.ShapeDtypeStruct((NPAD, H), jnp.float32),
    )(x, w)


def _two_mm_kernel(x_ref, wa_ref, wb_ref, oa_ref, ob_ref):
    x = x_ref[...]
    oa_ref[...] = jnp.dot(x, wa_ref[...], preferred_element_type=jnp.float32)
    ob_ref[...] = jnp.dot(x, wb_ref[...], preferred_element_type=jnp.float32)


def _two_mm(h, wa, wb):
    return _pcall(
        _two_mm_kernel,
        grid=(NBLK,),
        in_specs=[pl.BlockSpec((BN, H), lambda i: (i, 0)),
                  pl.BlockSpec((H, H), lambda i: (0, 0)),
                  pl.BlockSpec((H, H), lambda i: (0, 0))],
        out_specs=[pl.BlockSpec((BN, H), lambda i: (i, 0)),
                   pl.BlockSpec((BN, H), lambda i: (i, 0))],
        out_shape=[jax.ShapeDtypeStruct((NPAD, H), jnp.float32),
                   jax.ShapeDtypeStruct((NPAD, H), jnp.float32)],
    )(h, wa, wb)


def _knn_kernel(pd_ref, psT_ref, bd_ref, bs_ref, idx_ref, d_ref):
    i = pl.program_id(0)
    pd = pd_ref[...]
    psT = psT_ref[...]
    d2 = (jnp.sum(pd * pd, axis=1, keepdims=True)
          + jnp.sum(psT * psT, axis=0, keepdims=True)
          - 2.0 * jnp.dot(pd, psT, preferred_element_type=jnp.float32))
    src_iota = lax.broadcasted_iota(jnp.int32, (BN, NPAD), 1)
    dst_ids = i * BN + lax.broadcasted_iota(jnp.int32, (BN, NPAD), 0)
    bad = (bd_ref[...] != bs_ref[...]) | (dst_ids == src_iota)
    d2 = jnp.where(bad, jnp.inf, d2)
    for t in range(K):
        mn = jnp.min(d2, axis=1, keepdims=True)
        cand = jnp.where(d2 == mn, src_iota, NPAD + 1)
        sel = jnp.min(cand, axis=1, keepdims=True)
        idx_ref[:, t:t + 1] = sel
        d_ref[:, t:t + 1] = jnp.sqrt(jnp.maximum(mn, 0.0) + 1e-12)
        d2 = jnp.where(src_iota == sel, jnp.inf, d2)


def _knn(pos, psT, bcol, brow):
    return _pcall(
        _knn_kernel,
        grid=(NBLK,),
        in_specs=[pl.BlockSpec((BN, 3), lambda i: (i, 0)),
                  pl.BlockSpec((3, NPAD), lambda i: (0, 0)),
                  pl.BlockSpec((BN, 1), lambda i: (i, 0)),
                  pl.BlockSpec((1, NPAD), lambda i: (0, 0))],
        out_specs=[pl.BlockSpec((BN, K), lambda i: (i, 0)),
                   pl.BlockSpec((BN, K), lambda i: (i, 0))],
        out_shape=[jax.ShapeDtypeStruct((NPAD, K), jnp.int32),
                   jax.ShapeDtypeStruct((NPAD, K), jnp.float32)],
    )(pos, psT, bcol, brow)


_SC_NC = 2
_SC_NW = 32
_SC_CH = 128
_PER_W = EPAD // _SC_NW
_SC_NCH = _PER_W // _SC_CH


@functools.cache
def _build_sc_gather():
    @functools.partial(
        pl.kernel,
        mesh=plsc.VectorSubcoreMesh(core_axis_name="c", subcore_axis_name="s"),
        out_type=jax.ShapeDtypeStruct((EPAD, H), jnp.float32),
        scratch_types=[
            pltpu.VMEM((2, _SC_CH), jnp.int32),
            pltpu.VMEM((2, _SC_CH, H), jnp.float32),
            pltpu.SemaphoreType.DMA((2,)),
        ],
    )
    def _sc_gather(table_hbm, idx_hbm, out_hbm, idx_v, rows_v, sem):
        wid = lax.axis_index("s") * _SC_NC + lax.axis_index("c")
        base = wid * _PER_W

        pltpu.sync_copy(idx_hbm.at[pl.ds(base, _SC_CH)], idx_v.at[0])
        pltpu.make_async_copy(
            table_hbm.at[idx_v.at[0]], rows_v.at[0], sem.at[0]).start()

        def body(i, carry):
            b = lax.rem(i, 2)
            nb = lax.rem(i + 1, 2)

            @pl.when(i + 1 < _SC_NCH)
            def _():
                noff = base + (i + 1) * _SC_CH
                pltpu.sync_copy(idx_hbm.at[pl.ds(noff, _SC_CH)],
                                idx_v.at[nb])
                pltpu.make_async_copy(
                    table_hbm.at[idx_v.at[nb]], rows_v.at[nb],
                    sem.at[nb]).start()

            pltpu.make_async_copy(
                table_hbm.at[idx_v.at[b]], rows_v.at[b], sem.at[b]).wait()
            pltpu.sync_copy(rows_v.at[b],
                            out_hbm.at[pl.ds(base + i * _SC_CH, _SC_CH)])
            return carry

        lax.fori_loop(0, _SC_NCH, body, 0)

    return _sc_gather


def _GATHER(table, idx):
    return _build_sc_gather()(table, idx)


def _edge_kernel(h_ref, hA_ref, g_ref, d_ref,
                 c_ref, be1_ref, w2_ref, be2_ref,
                 wn1h_ref, wn1a_ref, bn1_ref, wn2_ref, bn2_ref,
                 lng_ref, lnb_ref, o_ref):
    d = d_ref[...]
    off = lax.broadcasted_iota(jnp.int32, (1, NUM_RBF), 1).astype(
        jnp.float32) * _RBF_STEP
    rbf = jnp.exp(_RBF_COEFF * (d - off) ** 2)
    cut = (d <= CUTOFF).astype(jnp.float32)
    hA = hA_ref[...]
    hA_e = jnp.reshape(jnp.broadcast_to(hA[:, None, :], (BN, K, H)),
                       (BN * K, H))
    m1 = jnp.maximum(
        hA_e + g_ref[...]
        + jnp.dot(rbf, c_ref[...], preferred_element_type=jnp.float32)
        + be1_ref[...], 0.0)
    m = jnp.maximum(
        jnp.dot(m1, w2_ref[...], preferred_element_type=jnp.float32)
        + be2_ref[...], 0.0) * cut
    agg = jnp.sum(jnp.reshape(m, (BN, K, H)), axis=1)
    h = h_ref[...]
    u1 = jnp.maximum(
        jnp.dot(h, wn1h_ref[...], preferred_element_type=jnp.float32)
        + jnp.dot(agg, wn1a_ref[...], preferred_element_type=jnp.float32)
        + bn1_ref[...], 0.0)
    u = jnp.dot(u1, wn2_ref[...], preferred_element_type=jnp.float32) \
        + bn2_ref[...]
    r = h + u
    mu = jnp.mean(r, axis=1, keepdims=True)
    var = jnp.mean((r - mu) ** 2, axis=1, keepdims=True)
    o_ref[...] = (r - mu) * lax.rsqrt(var + 1e-5) * lng_ref[...] \
        + lnb_ref[...]


def _edge_node(h, hA, g, dcol, c, be1, w2, be2,
               wn1h, wn1a, bn1, wn2, bn2, lng, lnb):
    full = lambda a: pl.BlockSpec(a.shape, lambda i: (0, 0))
    return _pcall(
        _edge_kernel,
        grid=(NBLK,),
        in_specs=[pl.BlockSpec((BN, H), lambda i: (i, 0)),
                  pl.BlockSpec((BN, H), lambda i: (i, 0)),
                  pl.BlockSpec((BN * K, H), lambda i: (i, 0)),
                  pl.BlockSpec((BN * K, 1), lambda i: (i, 0)),
                  full(c), full(be1), full(w2), full(be2),
                  full(wn1h), full(wn1a), full(bn1), full(wn2), full(bn2),
                  full(lng), full(lnb)],
        out_specs=pl.BlockSpec((BN, H), lambda i: (i, 0)),
        out_shape=jax.ShapeDtypeStruct((NPAD, H), jnp.float32),
    )(h, hA, g, dcol, c, be1, w2, be2, wn1h, wn1a, bn1, wn2, bn2, lng, lnb)


def _final_kernel(h_ref, enc_ref, brow_ref, w1h_ref, w1e_ref, b1_ref,
                  w2_ref, b2_ref, wo1h_ref, wo1g_ref, bo1_ref,
                  wo2_ref, bo2_ref, encg_ref, kind_ref, o_ref, acc_ref):
    i = pl.program_id(0)
    x1 = jnp.maximum(
        jnp.dot(h_ref[...], w1h_ref[...], preferred_element_type=jnp.float32)
        + jnp.dot(enc_ref[...], w1e_ref[...],
                  preferred_element_type=jnp.float32)
        + b1_ref[...], 0.0)
    x2 = jnp.dot(x1, w2_ref[...], preferred_element_type=jnp.float32) \
        + b2_ref[...]
    s_iota = lax.broadcasted_iota(jnp.int32, (NGRAPH, BN), 0)
    oh = (brow_ref[...] == s_iota).astype(jnp.float32)
    part = jnp.dot(oh, x2, preferred_element_type=jnp.float32)

    @pl.when(i == 0)
    def _():
        acc_ref[...] = jnp.zeros_like(acc_ref)

    acc_ref[...] += part

    @pl.when(i == pl.num_programs(0) - 1)
    def _():
        pre = acc_ref[...]
        z = (jnp.dot(pre, wo1h_ref[...], preferred_element_type=jnp.float32)
             + jnp.dot(encg_ref[...], wo1g_ref[...],
                       preferred_element_type=jnp.float32)
             + bo1_ref[...])
        sp = jnp.maximum(z, 0.0) + jnp.log1p(jnp.exp(-jnp.abs(z))) \
            - jnp.log(2.0).astype(jnp.float32)
        o2 = jnp.dot(sp, wo2_ref[...], preferred_element_type=jnp.float32) \
            + bo2_ref[...]
        k_oh = (kind_ref[...] - 1
                == lax.broadcasted_iota(jnp.int32, (NGRAPH, OUT_DIM), 1)
                ).astype(jnp.float32)
        o_ref[...] = jnp.sum(o2 * k_oh, axis=1, keepdims=True)


def _final(h, enc_pad, brow, w1h, w1e, b1, w2, b2,
           wo1h, wo1g, bo1, wo2, bo2, encg, kind):
    full = lambda a: pl.BlockSpec(a.shape, lambda i: (0, 0))
    return _pcall(
        _final_kernel,
        grid=(NBLK,),
        in_specs=[pl.BlockSpec((BN, H), lambda i: (i, 0)),
                  pl.BlockSpec((BN, enc_pad.shape[1]), lambda i: (i, 0)),
                  pl.BlockSpec((1, BN), lambda i: (0, i)),
                  full(w1h), full(w1e), full(b1), full(w2), full(b2),
                  full(wo1h), full(wo1g), full(bo1), full(wo2), full(bo2),
                  full(encg), full(kind)],
        out_specs=pl.BlockSpec((NGRAPH, 1), lambda i: (0, 0)),
        out_shape=jax.ShapeDtypeStruct((NGRAPH, 1), jnp.float32),
        scratch_shapes=[pltpu.VMEM((NGRAPH, H), jnp.float32)],
    )(h, enc_pad, brow, w1h, w1e, b1, w2, b2,
      wo1h, wo1g, bo1, wo2, bo2, encg, kind)


def kernel(protein_pos, protein_atom_feature, ligand_pos, ligand_atom_feature,
           batch_protein, batch_ligand, output_kind,
           enc_ligand_feature, enc_node_feature, enc_graph_feature, params):
    f32 = jnp.float32
    lig_feat = jnp.concatenate(
        [ligand_atom_feature, enc_ligand_feature], axis=1)
    pfd = protein_atom_feature.shape[1]
    lfd = lig_feat.shape[1]
    xa_d = 64
    xa = jnp.zeros((NPAD, xa_d), f32)
    xa = xa.at[:N_PROT, :pfd].set(protein_atom_feature)
    xa = xa.at[N_PROT:N_NODES, pfd:pfd + lfd].set(lig_feat)
    xa = xa.at[:N_PROT, pfd + lfd].set(1.0)
    xa = xa.at[N_PROT:N_NODES, pfd + lfd + 1].set(1.0)
    wa = jnp.zeros((xa_d, H), f32)
    wa = wa.at[:pfd].set(params['Wp'])
    wa = wa.at[pfd:pfd + lfd].set(params['Wl'])
    wa = wa.at[pfd + lfd].set(params['bp'])
    wa = wa.at[pfd + lfd + 1].set(params['bl'])

    pos = jnp.concatenate(
        [protein_pos, ligand_pos,
         jnp.zeros((NPAD - N_NODES, 3), f32)], axis=0)
    batch_all = jnp.concatenate(
        [batch_protein, batch_ligand]).astype(jnp.int32)
    batch_pad = jnp.concatenate(
        [batch_all, jnp.full((NPAD - N_NODES,), -1, jnp.int32)])
    brow = batch_pad.reshape(1, NPAD)
    bcol = batch_pad.reshape(NPAD, 1)
    psT = pos.T

    order = jnp.argsort(batch_all)
    inv = jnp.argsort(order)
    enc_aligned = enc_node_feature[inv]
    enc_pad = jnp.concatenate(
        [enc_aligned,
         jnp.zeros((NPAD - N_NODES, enc_aligned.shape[1]), f32)], axis=0)

    h = _proj(xa, wa)
    idx, dsel = _knn(pos, psT, bcol, brow)
    idx_flat = idx.reshape(EPAD)
    dcol = dsel.reshape(EPAD, 1)

    for lp in params['enc_layers']:
        a_w = lp['We1'][:H]
        b_w = lp['We1'][H:2 * H]
        c_w = lp['We1'][2 * H:]
        hA, hB = _two_mm(h, a_w, b_w)
        g = _GATHER(hB, idx_flat)
        h = _edge_node(h, hA, g, dcol, c_w,
                       lp['be1'].reshape(1, H), lp['We2'],
                       lp['be2'].reshape(1, H),
                       lp['Wn1'][:H], lp['Wn1'][H:],
                       lp['bn1'].reshape(1, H), lp['Wn2'],
                       lp['bn2'].reshape(1, H),
                       lp['ln_g'].reshape(1, H), lp['ln_b'].reshape(1, H))

    out = _final(
        h, enc_pad, brow,
        params['Wenc1'][:H], params['Wenc1'][H:],
        params['benc1'].reshape(1, H),
        params['Wenc2'], params['benc2'].reshape(1, H),
        params['Wo1'][:H], params['Wo1'][H:],
        params['bo1'].reshape(1, H),
        params['Wo2'], params['bo2'].reshape(1, OUT_DIM),
        enc_graph_feature.astype(f32),
        output_kind.reshape(NGRAPH, 1).astype(jnp.int32))
    return out
